# baseline (speedup 1.0000x reference)
import functools

import jax
import jax.numpy as jnp
from jax import lax
from jax.experimental import pallas as pl
from jax.experimental.pallas import tpu as pltpu
from jax.experimental.pallas import tpu_sc as plsc

_NC = 2
_NS = 16
_NW = _NC * _NS

_CHUNK_IDX = 100
_NBUF = 4
def _tc_pair(embT, V, D):
    BLK = 32768
    nblk = -(-V // BLK)

    def body(x_ref, o_ref):
        xt = jnp.transpose(x_ref[...]).astype(jnp.bfloat16)
        o_ref[...] = jnp.concatenate([xt[: BLK // 2], xt[BLK // 2 :]], axis=1)

    return pl.pallas_call(
        body,
        grid=(nblk,),
        in_specs=[pl.BlockSpec((D, BLK), lambda i: (0, i))],
        out_specs=pl.BlockSpec((BLK // 2, 2 * D), lambda i: (i, 0)),
        out_shape=jax.ShapeDtypeStruct((nblk * BLK // 2, 2 * D), jnp.bfloat16),
    )(embT)


def _sc_pool(idx3, emb, B, L, D):
    n_chunks = idx3.shape[1]
    rows_per_chunk = _CHUNK_IDX // L
    rows_per_w = n_chunks * rows_per_chunk
    nvec = D // 16
    inv_l = jnp.float32(1.0 / L)

    mesh = plsc.VectorSubcoreMesh(core_axis_name="c", subcore_axis_name="s")

    @functools.partial(
        pl.kernel,
        mesh=mesh,
        out_type=jax.ShapeDtypeStruct((B, D), jnp.float32),
        scratch_types=[
            pltpu.VMEM((n_chunks, _CHUNK_IDX), jnp.int32),
            pltpu.VMEM((_NBUF, _CHUNK_IDX, D), jnp.bfloat16),
            pltpu.VMEM((rows_per_w, D), jnp.float32),
            pltpu.SemaphoreType.DMA((_NBUF,)),
        ],
        compiler_params=pltpu.CompilerParams(
            use_tc_tiling_on_sc=False, needs_layout_passes=False
        ),
    )
    def sc_kernel(idx_hbm, emb_hbm, out_hbm, idx_v, rows_v, pooled_v, sems):
        w = lax.axis_index("s") * _NC + lax.axis_index("c")
        pltpu.sync_copy(idx_hbm.at[w], idx_v)

        for kslot in range(_NBUF):
            pltpu.make_async_copy(
                emb_hbm.at[idx_v.at[kslot]], rows_v.at[kslot], sems.at[kslot]
            ).start()

        def outer(g, carry):
            for kslot in range(_NBUF):
                t = g * _NBUF + kslot
                pltpu.make_async_copy(
                    emb_hbm.at[idx_v.at[t]], rows_v.at[kslot], sems.at[kslot]
                ).wait()
                for cr in range(rows_per_chunk):
                    accs = tuple(jnp.zeros((16,), jnp.float32) for _ in range(nvec))

                    mask_hi = jnp.int32(-65536)

                    def inner(i5, a, _k=kslot, _cr=cr):
                        base = _cr * L + i5 * 5
                        for u in range(5):
                            a = list(a)
                            for j in range(nvec // 2):
                                xw = rows_v[_k, base + u, pl.ds(j * 32, 32)]
                                xi = plsc.bitcast(xw, jnp.int32)
                                lo = plsc.bitcast(xi << 16, jnp.float32)
                                hi = plsc.bitcast(xi & mask_hi, jnp.float32)
                                a[2 * j] = a[2 * j] + lo
                                a[2 * j + 1] = a[2 * j + 1] + hi
                            a = tuple(a)
                        return a

                    accs = lax.fori_loop(0, L // 5, inner, accs)
                    row = t * rows_per_chunk + cr
                    for j in range(nvec):
                        pooled_v[row, pl.ds(j * 16, 16)] = accs[j] * inv_l
                nxt = t + _NBUF

                @pl.when(nxt < n_chunks)
                def _():
                    pltpu.make_async_copy(
                        emb_hbm.at[idx_v.at[nxt]], rows_v.at[kslot], sems.at[kslot]
                    ).start()
            return carry

        lax.fori_loop(0, n_chunks // _NBUF, outer, jnp.int32(0))
        pltpu.sync_copy(pooled_v, out_hbm.at[pl.ds(w * rows_per_w, rows_per_w)])

    return sc_kernel(idx3, emb)


def _tc_head(pooled, W, b, gamma, beta):
    B, D = pooled.shape

    def body(x_ref, w_ref, b_ref, g_ref, bt_ref, o_ref):
        x = x_ref[...]
        y = lax.dot_general(
            x, w_ref[...], (((1,), (1,)), ((), ())),
            preferred_element_type=jnp.float32,
            precision=lax.Precision.HIGHEST,
        ) + b_ref[...]
        mu = jnp.mean(y, axis=0, keepdims=True)
        var = jnp.mean((y - mu) ** 2, axis=0, keepdims=True)
        yn = (y - mu) * lax.rsqrt(var + 1e-5) * g_ref[...] + bt_ref[...]
        inv_norm = lax.rsqrt(jnp.sum(yn * yn, axis=1, keepdims=True))
        o_ref[...] = yn * inv_norm

    return pl.pallas_call(
        body,
        out_shape=jax.ShapeDtypeStruct((B, D), jnp.float32),
    )(pooled, W, b.reshape(1, D), gamma.reshape(1, D), beta.reshape(1, D))


def kernel(inputs, emb, W, b, gamma, beta):
    B, L = inputs.shape
    D = emb.shape[1]
    V = emb.shape[0]
    idx = inputs.astype(jnp.int32)
    blk = 32768
    q = idx & (blk - 1)
    idx = (idx & ~jnp.int32(blk - 1)) + ((q & (blk // 2 - 1)) << 1) + (q >> 14)
    n_per_w = (B // _NW) * L
    n_chunks = n_per_w // _CHUNK_IDX
    idx3 = idx.reshape(_NW, n_chunks, _CHUNK_IDX)
    embP = _tc_pair(emb.T, V, D)
    embL = embP.reshape(embP.shape[0] * 2, D)
    pooled = _sc_pool(idx3, embL, B, L, D)
    perm = sum(([list(range(g, g + 32, 2)), list(range(g + 1, g + 32, 2))]
                for g in range(0, D, 32)), [])
    p_idx = jnp.array([d for grp in perm for d in grp], dtype=jnp.int32)
    return _tc_head(pooled, W[:, p_idx], b, gamma, beta)


# speedup vs baseline: 1.6516x; 1.6516x over previous
import functools

import jax
import jax.numpy as jnp
from jax import lax
from jax.experimental import pallas as pl
from jax.experimental.pallas import tpu as pltpu
from jax.experimental.pallas import tpu_sc as plsc

_NC = 2
_NS = 16
_NW = _NC * _NS

_CHUNK_IDX = 100
_NBUF = 4
def _tc_pair(embT, V, D):
    BLK = 32768
    nblk = -(-V // BLK)

    Q = BLK // 4

    def body(x_ref, o_ref):
        xi = lax.bitcast_convert_type(x_ref[...], jnp.int32)
        rne = xi + jnp.int32(0x7FFF) + (lax.shift_right_logical(xi, 16) & 1)
        x3 = rne.reshape(D // 2, 2, BLK)
        lo = lax.shift_right_logical(x3[:, 0, :], 16)
        hi = x3[:, 1, :] & jnp.int32(-65536)
        packed = lo | hi
        pt = jnp.transpose(packed)
        o_ref[...] = jnp.concatenate(
            [pt[i * Q : (i + 1) * Q] for i in range(4)], axis=1)

    return pl.pallas_call(
        body,
        grid=(nblk,),
        in_specs=[pl.BlockSpec((D, BLK), lambda i: (0, i))],
        out_specs=pl.BlockSpec((Q, 2 * D), lambda i: (i, 0)),
        out_shape=jax.ShapeDtypeStruct((nblk * Q, 2 * D), jnp.int32),
    )(embT)


def _sc_pool(idx3, emb, B, L, D):
    n_chunks = idx3.shape[1]
    rows_per_chunk = _CHUNK_IDX // L
    rows_per_w = n_chunks * rows_per_chunk
    nvec = D // 16
    inv_l = jnp.float32(1.0 / L)

    mesh = plsc.VectorSubcoreMesh(core_axis_name="c", subcore_axis_name="s")

    @functools.partial(
        pl.kernel,
        mesh=mesh,
        out_type=jax.ShapeDtypeStruct((B, D), jnp.float32),
        scratch_types=[
            pltpu.VMEM((n_chunks, _CHUNK_IDX), jnp.int32),
            pltpu.VMEM((_NBUF, _CHUNK_IDX, D // 2), jnp.int32),
            pltpu.VMEM((rows_per_w, D), jnp.float32),
            pltpu.SemaphoreType.DMA((_NBUF,)),
        ],
        compiler_params=pltpu.CompilerParams(
            use_tc_tiling_on_sc=False, needs_layout_passes=False
        ),
    )
    def sc_kernel(idx_hbm, emb_hbm, out_hbm, idx_v, rows_v, pooled_v, sems):
        w = lax.axis_index("s") * _NC + lax.axis_index("c")
        pltpu.sync_copy(idx_hbm.at[w], idx_v)

        for kslot in range(_NBUF):
            pltpu.make_async_copy(
                emb_hbm.at[idx_v.at[kslot]], rows_v.at[kslot], sems.at[kslot]
            ).start()

        def outer(g, carry):
            for kslot in range(_NBUF):
                t = g * _NBUF + kslot
                pltpu.make_async_copy(
                    emb_hbm.at[idx_v.at[t]], rows_v.at[kslot], sems.at[kslot]
                ).wait()
                for cr in range(rows_per_chunk):
                    accs = tuple(jnp.zeros((16,), jnp.float32) for _ in range(nvec))

                    mask_hi = jnp.int32(-65536)

                    def inner(i5, a, _k=kslot, _cr=cr):
                        base = _cr * L + i5 * 5
                        for u in range(5):
                            a = list(a)
                            for j in range(nvec // 2):
                                xi = rows_v[_k, base + u, pl.ds(j * 16, 16)]
                                lo = plsc.bitcast(xi << 16, jnp.float32)
                                hi = plsc.bitcast(xi & mask_hi, jnp.float32)
                                a[2 * j] = a[2 * j] + lo
                                a[2 * j + 1] = a[2 * j + 1] + hi
                            a = tuple(a)
                        return a

                    accs = lax.fori_loop(0, L // 5, inner, accs)
                    row = t * rows_per_chunk + cr
                    for j in range(nvec):
                        pooled_v[row, pl.ds(j * 16, 16)] = accs[j] * inv_l
                nxt = t + _NBUF

                @pl.when(nxt < n_chunks)
                def _():
                    pltpu.make_async_copy(
                        emb_hbm.at[idx_v.at[nxt]], rows_v.at[kslot], sems.at[kslot]
                    ).start()
            return carry

        lax.fori_loop(0, n_chunks // _NBUF, outer, jnp.int32(0))
        pltpu.sync_copy(pooled_v, out_hbm.at[pl.ds(w * rows_per_w, rows_per_w)])

    return sc_kernel(idx3, emb)


def _tc_head(pooled, W, b, gamma, beta):
    B, D = pooled.shape

    def body(x_ref, w_ref, b_ref, g_ref, bt_ref, o_ref):
        x = x_ref[...]
        y = lax.dot_general(
            x, w_ref[...], (((1,), (1,)), ((), ())),
            preferred_element_type=jnp.float32,
            precision=lax.Precision.HIGHEST,
        ) + b_ref[...]
        mu = jnp.mean(y, axis=0, keepdims=True)
        var = jnp.mean((y - mu) ** 2, axis=0, keepdims=True)
        yn = (y - mu) * lax.rsqrt(var + 1e-5) * g_ref[...] + bt_ref[...]
        inv_norm = lax.rsqrt(jnp.sum(yn * yn, axis=1, keepdims=True))
        o_ref[...] = yn * inv_norm

    return pl.pallas_call(
        body,
        out_shape=jax.ShapeDtypeStruct((B, D), jnp.float32),
    )(pooled, W, b.reshape(1, D), gamma.reshape(1, D), beta.reshape(1, D))


def kernel(inputs, emb, W, b, gamma, beta):
    B, L = inputs.shape
    D = emb.shape[1]
    V = emb.shape[0]
    idx = inputs.astype(jnp.int32)
    blk = 32768
    q = idx & (blk - 1)
    idx = (idx & ~jnp.int32(blk - 1)) + ((q & (blk // 4 - 1)) << 2) + (q >> 13)
    n_per_w = (B // _NW) * L
    n_chunks = n_per_w // _CHUNK_IDX
    idx3 = idx.reshape(_NW, n_chunks, _CHUNK_IDX)
    embP = _tc_pair(emb.T, V, D)
    embL = embP.reshape(embP.shape[0] * 4, D // 2)
    pooled = _sc_pool(idx3, embL, B, L, D)
    perm = sum(([list(range(g, g + 32, 2)), list(range(g + 1, g + 32, 2))]
                for g in range(0, D, 32)), [])
    p_idx = jnp.array([d for grp in perm for d in grp], dtype=jnp.int32)
    return _tc_head(pooled, W[:, p_idx], b, gamma, beta)


# speedup vs baseline: 1.7932x; 1.0857x over previous
import functools

import jax
import jax.numpy as jnp
from jax import lax
from jax.experimental import pallas as pl
from jax.experimental.pallas import tpu as pltpu
from jax.experimental.pallas import tpu_sc as plsc

_NC = 2
_NS = 16
_NW = _NC * _NS

_CHUNK_IDX = 100
_NBUF = 4
def _tc_pair(embT, V, D):
    BLK = 32768
    nblk = -(-V // BLK)

    Q = BLK // 4

    def body(x_ref, o_ref):
        xi = lax.bitcast_convert_type(x_ref[...], jnp.int32)
        rne = xi + jnp.int32(0x7FFF) + (lax.shift_right_logical(xi, 16) & 1)
        lo = lax.shift_right_logical(rne[: D // 2], 16)
        hi = rne[D // 2 :] & jnp.int32(-65536)
        packed = lo | hi
        pt = jnp.transpose(packed)
        o_ref[...] = jnp.concatenate(
            [pt[i * Q : (i + 1) * Q] for i in range(4)], axis=1)

    return pl.pallas_call(
        body,
        grid=(nblk,),
        in_specs=[pl.BlockSpec((D, BLK), lambda i: (0, i))],
        out_specs=pl.BlockSpec((Q, 2 * D), lambda i: (i, 0)),
        out_shape=jax.ShapeDtypeStruct((nblk * Q, 2 * D), jnp.int32),
    )(embT)


def _sc_pool(idx3, emb, B, L, D):
    n_chunks = idx3.shape[1]
    rows_per_chunk = _CHUNK_IDX // L
    rows_per_w = n_chunks * rows_per_chunk
    nvec = D // 16
    inv_l = jnp.float32(1.0 / L)

    mesh = plsc.VectorSubcoreMesh(core_axis_name="c", subcore_axis_name="s")

    @functools.partial(
        pl.kernel,
        mesh=mesh,
        out_type=jax.ShapeDtypeStruct((B, D), jnp.float32),
        scratch_types=[
            pltpu.VMEM((n_chunks, _CHUNK_IDX), jnp.int32),
            pltpu.VMEM((_NBUF, _CHUNK_IDX, D // 2), jnp.int32),
            pltpu.VMEM((rows_per_w, D), jnp.float32),
            pltpu.SemaphoreType.DMA((_NBUF,)),
        ],
        compiler_params=pltpu.CompilerParams(
            use_tc_tiling_on_sc=False, needs_layout_passes=False
        ),
    )
    def sc_kernel(idx_hbm, emb_hbm, out_hbm, idx_v, rows_v, pooled_v, sems):
        w = lax.axis_index("s") * _NC + lax.axis_index("c")
        pltpu.sync_copy(idx_hbm.at[w], idx_v)

        for kslot in range(_NBUF):
            pltpu.make_async_copy(
                emb_hbm.at[idx_v.at[kslot]], rows_v.at[kslot], sems.at[kslot]
            ).start()

        def outer(g, carry):
            for kslot in range(_NBUF):
                t = g * _NBUF + kslot
                pltpu.make_async_copy(
                    emb_hbm.at[idx_v.at[t]], rows_v.at[kslot], sems.at[kslot]
                ).wait()
                for cr in range(rows_per_chunk):
                    accs = tuple(jnp.zeros((16,), jnp.float32) for _ in range(nvec))

                    mask_hi = jnp.int32(-65536)

                    def inner(i5, a, _k=kslot, _cr=cr):
                        base = _cr * L + i5 * 5
                        for u in range(5):
                            a = list(a)
                            for j in range(nvec // 2):
                                xi = rows_v[_k, base + u, pl.ds(j * 16, 16)]
                                lo = plsc.bitcast(xi << 16, jnp.float32)
                                hi = plsc.bitcast(xi & mask_hi, jnp.float32)
                                a[2 * j] = a[2 * j] + lo
                                a[2 * j + 1] = a[2 * j + 1] + hi
                            a = tuple(a)
                        return a

                    accs = lax.fori_loop(0, L // 5, inner, accs)
                    row = t * rows_per_chunk + cr
                    for j in range(nvec):
                        pooled_v[row, pl.ds(j * 16, 16)] = accs[j] * inv_l
                nxt = t + _NBUF

                @pl.when(nxt < n_chunks)
                def _():
                    pltpu.make_async_copy(
                        emb_hbm.at[idx_v.at[nxt]], rows_v.at[kslot], sems.at[kslot]
                    ).start()
            return carry

        lax.fori_loop(0, n_chunks // _NBUF, outer, jnp.int32(0))
        pltpu.sync_copy(pooled_v, out_hbm.at[pl.ds(w * rows_per_w, rows_per_w)])

    return sc_kernel(idx3, emb)


def _tc_head(pooled, W, b, gamma, beta):
    B, D = pooled.shape

    def body(x_ref, w_ref, b_ref, g_ref, bt_ref, o_ref):
        x = x_ref[...]
        y = lax.dot_general(
            x, w_ref[...], (((1,), (1,)), ((), ())),
            preferred_element_type=jnp.float32,
            precision=lax.Precision.HIGHEST,
        ) + b_ref[...]
        mu = jnp.mean(y, axis=0, keepdims=True)
        var = jnp.mean((y - mu) ** 2, axis=0, keepdims=True)
        yn = (y - mu) * lax.rsqrt(var + 1e-5) * g_ref[...] + bt_ref[...]
        inv_norm = lax.rsqrt(jnp.sum(yn * yn, axis=1, keepdims=True))
        o_ref[...] = yn * inv_norm

    return pl.pallas_call(
        body,
        out_shape=jax.ShapeDtypeStruct((B, D), jnp.float32),
    )(pooled, W, b.reshape(1, D), gamma.reshape(1, D), beta.reshape(1, D))


def kernel(inputs, emb, W, b, gamma, beta):
    B, L = inputs.shape
    D = emb.shape[1]
    V = emb.shape[0]
    idx = inputs.astype(jnp.int32)
    blk = 32768
    q = idx & (blk - 1)
    idx = (idx & ~jnp.int32(blk - 1)) + ((q & (blk // 4 - 1)) << 2) + (q >> 13)
    n_per_w = (B // _NW) * L
    n_chunks = n_per_w // _CHUNK_IDX
    idx3 = idx.reshape(_NW, n_chunks, _CHUNK_IDX)
    embP = _tc_pair(emb.T, V, D)
    embL = embP.reshape(embP.shape[0] * 4, D // 2)
    pooled = _sc_pool(idx3, embL, B, L, D)
    h = D // 2
    perm = sum(([list(range(j * 16, j * 16 + 16)),
                 list(range(h + j * 16, h + j * 16 + 16))]
                for j in range(h // 16)), [])
    p_idx = jnp.array([d for grp in perm for d in grp], dtype=jnp.int32)
    return _tc_head(pooled, W[:, p_idx], b, gamma, beta)


# speedup vs baseline: 1.8997x; 1.0594x over previous
import functools

import jax
import jax.numpy as jnp
from jax import lax
from jax.experimental import pallas as pl
from jax.experimental.pallas import tpu as pltpu
from jax.experimental.pallas import tpu_sc as plsc

_NC = 2
_NS = 16
_NW = _NC * _NS

_CHUNK_IDX = 100
_NBUF = 8
def _tc_pair(embT, V, D):
    BLK = 32768
    nblk = -(-V // BLK)

    Q = BLK // 4

    def body(x_ref, o_ref):
        xi = lax.bitcast_convert_type(x_ref[...], jnp.int32)
        lo = lax.shift_right_logical(xi[: D // 2], 16)
        hi = xi[D // 2 :] & jnp.int32(-65536)
        packed = lo | hi
        pt = jnp.transpose(packed)
        o_ref[...] = jnp.concatenate(
            [pt[i * Q : (i + 1) * Q] for i in range(4)], axis=1)

    return pl.pallas_call(
        body,
        grid=(nblk,),
        in_specs=[pl.BlockSpec((D, BLK), lambda i: (0, i))],
        out_specs=pl.BlockSpec((Q, 2 * D), lambda i: (i, 0)),
        out_shape=jax.ShapeDtypeStruct((nblk * Q, 2 * D), jnp.int32),
    )(embT)


def _sc_pool(idx3, emb, B, L, D):
    n_chunks = idx3.shape[1]
    rows_per_chunk = _CHUNK_IDX // L
    rows_per_w = n_chunks * rows_per_chunk
    nvec = D // 16
    inv_l = jnp.float32(1.0 / L)

    mesh = plsc.VectorSubcoreMesh(core_axis_name="c", subcore_axis_name="s")

    @functools.partial(
        pl.kernel,
        mesh=mesh,
        out_type=jax.ShapeDtypeStruct((B, D), jnp.float32),
        scratch_types=[
            pltpu.VMEM((n_chunks, _CHUNK_IDX), jnp.int32),
            pltpu.VMEM((_NBUF, _CHUNK_IDX, D // 2), jnp.int32),
            pltpu.VMEM((rows_per_w, D), jnp.float32),
            pltpu.SemaphoreType.DMA((_NBUF,)),
        ],
        compiler_params=pltpu.CompilerParams(
            use_tc_tiling_on_sc=False, needs_layout_passes=False
        ),
    )
    def sc_kernel(idx_hbm, emb_hbm, out_hbm, idx_v, rows_v, pooled_v, sems):
        w = lax.axis_index("s") * _NC + lax.axis_index("c")
        pltpu.sync_copy(idx_hbm.at[w], idx_v)

        for kslot in range(_NBUF):
            pltpu.make_async_copy(
                emb_hbm.at[idx_v.at[kslot]], rows_v.at[kslot], sems.at[kslot]
            ).start()

        def outer(g, carry):
            for kslot in range(_NBUF):
                t = g * _NBUF + kslot
                pltpu.make_async_copy(
                    emb_hbm.at[idx_v.at[t]], rows_v.at[kslot], sems.at[kslot]
                ).wait()
                for cr in range(rows_per_chunk):
                    accs = tuple(jnp.zeros((16,), jnp.float32) for _ in range(nvec))

                    mask_hi = jnp.int32(-65536)

                    def inner(i5, a, _k=kslot, _cr=cr):
                        base = _cr * L + i5 * 5
                        for u in range(5):
                            a = list(a)
                            for j in range(nvec // 2):
                                xi = rows_v[_k, base + u, pl.ds(j * 16, 16)]
                                lo = plsc.bitcast(xi << 16, jnp.float32)
                                hi = plsc.bitcast(xi & mask_hi, jnp.float32)
                                a[2 * j] = a[2 * j] + lo
                                a[2 * j + 1] = a[2 * j + 1] + hi
                            a = tuple(a)
                        return a

                    accs = lax.fori_loop(0, L // 5, inner, accs)
                    row = t * rows_per_chunk + cr
                    for j in range(nvec):
                        pooled_v[row, pl.ds(j * 16, 16)] = accs[j] * inv_l
                nxt = t + _NBUF

                @pl.when(nxt < n_chunks)
                def _():
                    pltpu.make_async_copy(
                        emb_hbm.at[idx_v.at[nxt]], rows_v.at[kslot], sems.at[kslot]
                    ).start()
            return carry

        lax.fori_loop(0, n_chunks // _NBUF, outer, jnp.int32(0))
        pltpu.sync_copy(pooled_v, out_hbm.at[pl.ds(w * rows_per_w, rows_per_w)])

    return sc_kernel(idx3, emb)


def _tc_head(pooled, W, b, gamma, beta):
    B, D = pooled.shape

    def body(x_ref, w_ref, b_ref, g_ref, bt_ref, o_ref):
        x = x_ref[...]
        y = lax.dot_general(
            x, w_ref[...], (((1,), (1,)), ((), ())),
            preferred_element_type=jnp.float32,
        ) + b_ref[...]
        mu = jnp.mean(y, axis=0, keepdims=True)
        var = jnp.mean((y - mu) ** 2, axis=0, keepdims=True)
        yn = (y - mu) * lax.rsqrt(var + 1e-5) * g_ref[...] + bt_ref[...]
        inv_norm = lax.rsqrt(jnp.sum(yn * yn, axis=1, keepdims=True))
        o_ref[...] = yn * inv_norm

    return pl.pallas_call(
        body,
        out_shape=jax.ShapeDtypeStruct((B, D), jnp.float32),
    )(pooled, W, b.reshape(1, D), gamma.reshape(1, D), beta.reshape(1, D))


def kernel(inputs, emb, W, b, gamma, beta):
    B, L = inputs.shape
    D = emb.shape[1]
    V = emb.shape[0]
    idx = inputs.astype(jnp.int32)
    blk = 32768
    q = idx & (blk - 1)
    idx = (idx & ~jnp.int32(blk - 1)) + ((q & (blk // 4 - 1)) << 2) + (q >> 13)
    n_per_w = (B // _NW) * L
    n_chunks = n_per_w // _CHUNK_IDX
    idx3 = idx.reshape(_NW, n_chunks, _CHUNK_IDX)
    embP = _tc_pair(emb.T, V, D)
    embL = embP.reshape(embP.shape[0] * 4, D // 2)
    pooled = _sc_pool(idx3, embL, B, L, D)
    h = D // 2
    perm = sum(([list(range(j * 16, j * 16 + 16)),
                 list(range(h + j * 16, h + j * 16 + 16))]
                for j in range(h // 16)), [])
    p_idx = jnp.array([d for grp in perm for d in grp], dtype=jnp.int32)
    return _tc_head(pooled, W[:, p_idx], b, gamma, beta)
